# baseline (speedup 1.0000x reference)
import jax
import jax.numpy as jnp
from jax import lax
from jax.experimental import pallas as pl
from jax.experimental.pallas import tpu as pltpu
from jax.experimental.pallas import tpu_sc as plsc

VOCAB = 1_000_000
EMB = 64
BATCH = 16384
SEQ = 50

NC = 2
NS = 16
NW = NC * NS
CH = 128
BPW = BATCH // NW
KPW = BPW // CH
NSLAB = SEQ * KPW
D = 2
NG = 4
NT = 2


def _body(idx_hbm, w_hbm, out_hbm,
          stage_v, tstage_v,
          idx0, idx1, idx2, idx3,
          gsem, ssem, isem):
    wid = lax.axis_index("s") * NC + lax.axis_index("c")
    bbase = wid * BPW
    idx_ring = [idx0, idx1, idx2, idx3]

    rows16 = [lax.iota(jnp.int32, 16) + 16 * i for i in range(8)]

    def idx_fetch(s, slot):
        l = s // KPW
        b0 = bbase + (s % KPW) * CH
        return pltpu.make_async_copy(
            idx_hbm.at[l, pl.ds(b0, CH)], idx_ring[slot], isem.at[slot])

    def gather_desc(slot):
        return pltpu.make_async_copy(
            w_hbm.at[idx_ring[slot]], stage_v.at[slot], gsem.at[slot])

    def store_desc(s, slot):
        l = s // KPW
        tc = (s % KPW) * CH + bbase
        return pltpu.make_async_copy(
            tstage_v.at[slot],
            out_hbm.at[l, :, tc // CH, :, :], ssem.at[slot])

    def prepare(s, slot):
        idx_fetch(s, slot).wait()
        gather_desc(slot).start()

    def transpose(gslot, tslot):
        st = stage_v.at[gslot]
        ts = tstage_v.at[tslot]
        zero = lax.shift_right_logical(idx_ring[gslot][pl.ds(0, 16)], 31)
        rows_rt = [rows16[i] + zero for i in range(8)]
        col = zero
        prev = None
        for e in range(EMB):
            cur = []
            for i in range(8):
                cur.append(plsc.load_gather(st, [rows_rt[i], col]))
                if prev is not None:
                    ts[(e - 1) >> 3, (e - 1) & 7, pl.ds(16 * i, 16)] = prev[i]
            if e + 1 < EMB:
                col = col + 1
            prev = cur
        for i in range(8):
            ts[(EMB - 1) >> 3, (EMB - 1) & 7, pl.ds(16 * i, 16)] = prev[i]

    for s in range(NG):
        idx_fetch(s, s).start()
    for s in range(D):
        prepare(s, s)

    @pl.loop(0, NSLAB // NG)
    def _(g):
        for b in range(NG):
            s = g * NG + b
            pslot = (b + D) % NG
            tslot = b % NT

            @pl.when(s + D < NSLAB)
            def _():
                prepare(s + D, pslot)

            gather_desc(b).wait()

            @pl.when(s + NG < NSLAB)
            def _():
                idx_fetch(s + NG, b).start()

            @pl.when(s >= NT)
            def _():
                store_desc(s - NT, tslot).wait()

            transpose(b, tslot)
            store_desc(s, tslot).start()

    for s in range(NSLAB - NT, NSLAB):
        store_desc(s, s % NT).wait()


@jax.jit
def _gather(idxT, weight):
    mesh = plsc.VectorSubcoreMesh(core_axis_name="c", subcore_axis_name="s")
    return pl.kernel(
        _body,
        out_type=jax.ShapeDtypeStruct((SEQ, 8, BATCH // CH, 8, CH),
                                      jnp.float32),
        mesh=mesh,
        scratch_types=(
            [pltpu.VMEM((NG, CH, EMB), jnp.float32),
             pltpu.VMEM((NT, 8, 8, CH), jnp.float32)]
            + [pltpu.VMEM((CH,), jnp.int32) for _ in range(NG)]
            + [pltpu.SemaphoreType.DMA((NG,)),
               pltpu.SemaphoreType.DMA((NT,)),
               pltpu.SemaphoreType.DMA((NG,))]
        ),
        compiler_params=pltpu.CompilerParams(
            use_tc_tiling_on_sc=False, needs_layout_passes=False),
    )(idxT, weight)


def kernel(input, weight):
    idxT = input.T
    out5 = _gather(idxT, weight)
    out = jnp.transpose(out5, (2, 4, 0, 1, 3)).reshape(BATCH, SEQ, EMB)
    return out


# speedup vs baseline: 1.5393x; 1.5393x over previous
import jax
import jax.numpy as jnp
from jax import lax
from jax.experimental import pallas as pl
from jax.experimental.pallas import tpu as pltpu
from jax.experimental.pallas import tpu_sc as plsc

VOCAB = 1_000_000
EMB = 64
BATCH = 16384
SEQ = 50

NC = 2
NS = 16
NW = NC * NS
CH = 128
BPW = BATCH // NW
KPW = BPW // CH
NSLAB = SEQ * KPW
D = 2
NG = 4
NT = 2


def _body(idx_hbm, w_hbm, out_hbm,
          stage_v, tstage_v,
          idx0, idx1, idx2, idx3,
          gsem, ssem, isem):
    wid = lax.axis_index("s") * NC + lax.axis_index("c")
    bbase = wid * BPW
    idx_ring = [idx0, idx1, idx2, idx3]

    rows16 = [lax.iota(jnp.int32, 16) + 16 * i for i in range(8)]

    def idx_fetch(s, slot):
        l = s // KPW
        b0 = bbase + (s % KPW) * CH
        return pltpu.make_async_copy(
            idx_hbm.at[l, pl.ds(b0, CH)], idx_ring[slot], isem.at[slot])

    def gather_desc(slot):
        return pltpu.make_async_copy(
            w_hbm.at[idx_ring[slot]], stage_v.at[slot], gsem.at[slot])

    def store_desc(s, slot):
        l = s // KPW
        tc = (s % KPW) * CH + bbase
        return pltpu.make_async_copy(
            tstage_v.at[slot, :, :, pl.ds(0, CH)],
            out_hbm.at[l, :, tc // CH, :, :], ssem.at[slot])

    def prepare(s, slot):
        idx_fetch(s, slot).wait()
        gather_desc(slot).start()

    def transpose(gslot, tslot):
        st = stage_v.at[gslot]
        ts = tstage_v.at[tslot]
        zero = lax.shift_right_logical(idx_ring[gslot][pl.ds(0, 16)], 31)
        iota = lax.iota(jnp.int32, 16) + zero
        elo = lax.bitwise_and(iota, 7)
        ehi = [lax.shift_right_logical(iota, 3) + 2 * j for j in range(4)]
        col = zero
        pcol = None
        prev = None
        for r in range(CH):
            cur = [st[r, pl.ds(16 * j, 16)] for j in range(4)]
            if prev is not None:
                for j in range(4):
                    plsc.store_scatter(ts, [ehi[j], elo, pcol], prev[j])
            pcol = col
            if r + 1 < CH:
                col = col + 1
            prev = cur
        for j in range(4):
            plsc.store_scatter(ts, [ehi[j], elo, pcol], prev[j])

    for s in range(NG):
        idx_fetch(s, s).start()
    for s in range(D):
        prepare(s, s)

    @pl.loop(0, NSLAB // NG)
    def _(g):
        for b in range(NG):
            s = g * NG + b
            pslot = (b + D) % NG
            tslot = b % NT

            @pl.when(s + D < NSLAB)
            def _():
                prepare(s + D, pslot)

            gather_desc(b).wait()

            @pl.when(s + NG < NSLAB)
            def _():
                idx_fetch(s + NG, b).start()

            @pl.when(s >= NT)
            def _():
                store_desc(s - NT, tslot).wait()

            transpose(b, tslot)
            store_desc(s, tslot).start()

    for s in range(NSLAB - NT, NSLAB):
        store_desc(s, s % NT).wait()


@jax.jit
def _gather(idxT, weight):
    mesh = plsc.VectorSubcoreMesh(core_axis_name="c", subcore_axis_name="s")
    return pl.kernel(
        _body,
        out_type=jax.ShapeDtypeStruct((SEQ, 8, BATCH // CH, 8, CH),
                                      jnp.float32),
        mesh=mesh,
        scratch_types=(
            [pltpu.VMEM((NG, CH, EMB), jnp.float32),
             pltpu.VMEM((NT, 8, 8, CH + 1), jnp.float32)]
            + [pltpu.VMEM((CH,), jnp.int32) for _ in range(NG)]
            + [pltpu.SemaphoreType.DMA((NG,)),
               pltpu.SemaphoreType.DMA((NT,)),
               pltpu.SemaphoreType.DMA((NG,))]
        ),
        compiler_params=pltpu.CompilerParams(
            use_tc_tiling_on_sc=False, needs_layout_passes=False),
    )(idxT, weight)


def kernel(input, weight):
    idxT = input.T
    out5 = _gather(idxT, weight)
    out = jnp.transpose(out5, (2, 4, 0, 1, 3)).reshape(BATCH, SEQ, EMB)
    return out


# speedup vs baseline: 1.5441x; 1.0031x over previous
import jax
import jax.numpy as jnp
from jax import lax
from jax.experimental import pallas as pl
from jax.experimental.pallas import tpu as pltpu
from jax.experimental.pallas import tpu_sc as plsc

VOCAB = 1_000_000
EMB = 64
BATCH = 16384
SEQ = 50

NC = 2
NS = 16
NW = NC * NS
CH = 128
BPW = BATCH // NW
KPW = BPW // CH
NSLAB = SEQ * KPW
D = 3
NG = 4
NT = 2


def _body(idx_hbm, w_hbm, out_hbm,
          stage_v, tstage_v,
          idx0, idx1, idx2, idx3,
          gsem, ssem, isem):
    wid = lax.axis_index("s") * NC + lax.axis_index("c")
    bbase = wid * BPW
    idx_ring = [idx0, idx1, idx2, idx3]

    rows16 = [lax.iota(jnp.int32, 16) + 16 * i for i in range(8)]

    def idx_fetch(s, slot):
        l = s // KPW
        b0 = bbase + (s % KPW) * CH
        return pltpu.make_async_copy(
            idx_hbm.at[l, pl.ds(b0, CH)], idx_ring[slot], isem.at[slot])

    def gather_desc(slot):
        return pltpu.make_async_copy(
            w_hbm.at[idx_ring[slot]], stage_v.at[slot], gsem.at[slot])

    def store_desc(s, slot):
        l = s // KPW
        tc = (s % KPW) * CH + bbase
        return pltpu.make_async_copy(
            tstage_v.at[slot, :, :, pl.ds(0, CH)],
            out_hbm.at[l, :, tc // CH, :, :], ssem.at[slot])

    def prepare(s, slot):
        idx_fetch(s, slot).wait()
        gather_desc(slot).start()

    def transpose(gslot, tslot):
        st = stage_v.at[gslot]
        ts = tstage_v.at[tslot]
        zero = lax.shift_right_logical(idx_ring[gslot][pl.ds(0, 16)], 31)
        iota = lax.iota(jnp.int32, 16) + zero
        elo = lax.bitwise_and(iota, 7)
        ehi = [lax.shift_right_logical(iota, 3) + 2 * j for j in range(4)]
        col = zero
        pcol = None
        prev = None
        for r in range(CH):
            cur = [st[r, pl.ds(16 * j, 16)] for j in range(4)]
            if prev is not None:
                for j in range(4):
                    plsc.store_scatter(ts, [ehi[j], elo, pcol], prev[j])
            pcol = col
            if r + 1 < CH:
                col = col + 1
            prev = cur
        for j in range(4):
            plsc.store_scatter(ts, [ehi[j], elo, pcol], prev[j])

    for s in range(NG):
        idx_fetch(s, s).start()
    for s in range(D):
        prepare(s, s)

    @pl.loop(0, NSLAB // NG)
    def _(g):
        for b in range(NG):
            s = g * NG + b
            pslot = (b + D) % NG
            tslot = b % NT

            @pl.when(s + D < NSLAB)
            def _():
                prepare(s + D, pslot)

            gather_desc(b).wait()

            @pl.when(s + NG < NSLAB)
            def _():
                idx_fetch(s + NG, b).start()

            @pl.when(s >= NT)
            def _():
                store_desc(s - NT, tslot).wait()

            transpose(b, tslot)
            store_desc(s, tslot).start()

    for s in range(NSLAB - NT, NSLAB):
        store_desc(s, s % NT).wait()


@jax.jit
def _gather(idxT, weight):
    mesh = plsc.VectorSubcoreMesh(core_axis_name="c", subcore_axis_name="s")
    return pl.kernel(
        _body,
        out_type=jax.ShapeDtypeStruct((SEQ, 8, BATCH // CH, 8, CH),
                                      jnp.float32),
        mesh=mesh,
        scratch_types=(
            [pltpu.VMEM((NG, CH, EMB), jnp.float32),
             pltpu.VMEM((NT, 8, 8, CH + 1), jnp.float32)]
            + [pltpu.VMEM((CH,), jnp.int32) for _ in range(NG)]
            + [pltpu.SemaphoreType.DMA((NG,)),
               pltpu.SemaphoreType.DMA((NT,)),
               pltpu.SemaphoreType.DMA((NG,))]
        ),
        compiler_params=pltpu.CompilerParams(
            use_tc_tiling_on_sc=False, needs_layout_passes=False),
    )(idxT, weight)


def kernel(input, weight):
    idxT = input.T
    out5 = _gather(idxT, weight)
    out = jnp.transpose(out5, (2, 4, 0, 1, 3)).reshape(BATCH, SEQ, EMB)
    return out
